# baseline (speedup 1.0000x reference)
import jax
import jax.numpy as jnp
from jax.experimental import pallas as pl
from jax.experimental.pallas import tpu as pltpu

SCORE_THRESH = 0.05
NMS_THRESH = 0.5
DETECTIONS_PER_IMG = 300
TOPK_CANDIDATES = 1000
NEG = -1e9

K = 1024
P = 384


def _corners(cx, cy, w, h, t):
    c = jnp.abs(jnp.cos(t))
    s = jnp.abs(jnp.sin(t))
    W = w * c + h * s
    H = w * s + h * c
    return cx - W * 0.5, cy - H * 0.5, cx + W * 0.5, cy + H * 0.5


def _nms_kernel(acol_ref, arow_ref, o_ref, iou_ref):
    Ac = acol_ref[0]
    Ar = arow_ref[0]

    x1c, y1c, x2c, y2c = _corners(
        Ac[:, 0:1], Ac[:, 1:2], Ac[:, 2:3], Ac[:, 3:4], Ac[:, 4:5])
    area_c = (x2c - x1c) * (y2c - y1c)
    x1r, y1r, x2r, y2r = _corners(
        Ar[0:1, :], Ar[1:2, :], Ar[2:3, :], Ar[3:4, :], Ar[4:5, :])
    area_r = (x2r - x1r) * (y2r - y1r)

    xx1 = jnp.maximum(x1c, x1r)
    yy1 = jnp.maximum(y1c, y1r)
    xx2 = jnp.minimum(x2c, x2r)
    yy2 = jnp.minimum(y2c, y2r)
    inter = jnp.maximum(xx2 - xx1, 0.0) * jnp.maximum(yy2 - yy1, 0.0)
    union = area_c + area_r - inter
    iou = inter / (union + 1e-7)
    same = Ac[:, 6:7] == Ar[6:7, :]
    iou_ref[:, :] = jnp.where(same, iou, 0.0)

    colid = jax.lax.broadcasted_iota(jnp.int32, (1, K), 1)
    keep0 = jnp.ones((1, K), jnp.float32)

    def body(i, keep):
        row = iou_ref[pl.ds(i, 1), :]
        ki = jnp.sum(jnp.where(colid == i, keep, 0.0))
        sup = (row > NMS_THRESH) & (colid > i) & (ki > 0.5)
        return jnp.where(sup, 0.0, keep)

    keep = jax.lax.fori_loop(0, K, body, keep0)

    s_row = Ar[5:6, :]
    validf = keep * (s_row > NEG / 2).astype(jnp.float32)

    ri = jax.lax.broadcasted_iota(jnp.int32, (K, K), 0)
    ci = jax.lax.broadcasted_iota(jnp.int32, (K, K), 1)
    tri = (ri <= ci).astype(jnp.float32)
    cum = jnp.dot(validf, tri, preferred_element_type=jnp.float32)
    pos = cum - 1.0

    prow = jax.lax.broadcasted_iota(jnp.int32, (P, 1), 0).astype(jnp.float32)
    onehot = ((prow == pos) & (validf > 0.5)).astype(jnp.float32)

    lab_p1 = Ac[:, 6:7] + 1.0
    vals = jnp.concatenate([Ac[:, 0:6], lab_p1, Ac[:, 7:8]], axis=1)
    out = jnp.dot(onehot, vals, preferred_element_type=jnp.float32)
    chan = jax.lax.broadcasted_iota(jnp.int32, (P, 8), 1)
    out = jnp.where(chan == 6, out - 1.0, out)
    o_ref[0] = out


def kernel(boxes, scores, labels):
    B, N = scores.shape
    scores_f = jnp.where(scores > SCORE_THRESH, scores, NEG)
    top_s, idx = jax.lax.top_k(scores_f, TOPK_CANDIDATES)
    top_b = jnp.take_along_axis(boxes, idx[..., None], axis=1)
    top_l = jnp.take_along_axis(labels, idx, axis=1)

    pad = K - TOPK_CANDIDATES
    top_s = jnp.pad(top_s, ((0, 0), (0, pad)), constant_values=NEG)
    top_b = jnp.pad(top_b, ((0, 0), (0, pad), (0, 0)))
    top_l = jnp.pad(top_l, ((0, 0), (0, pad)), constant_values=-1)

    A = jnp.concatenate(
        [top_b, top_s[..., None], top_l[..., None].astype(jnp.float32),
         jnp.zeros((B, K, 1), jnp.float32)], axis=-1)
    At = jnp.swapaxes(A, 1, 2)

    O = pl.pallas_call(
        _nms_kernel,
        grid=(B,),
        in_specs=[
            pl.BlockSpec((1, K, 8), lambda b: (b, 0, 0)),
            pl.BlockSpec((1, 8, K), lambda b: (b, 0, 0)),
        ],
        out_specs=pl.BlockSpec((1, P, 8), lambda b: (b, 0, 0)),
        out_shape=jax.ShapeDtypeStruct((B, P, 8), jnp.float32),
        scratch_shapes=[pltpu.VMEM((K, K), jnp.float32)],
        compiler_params=pltpu.CompilerParams(
            dimension_semantics=("parallel",)),
    )(A, At)

    out_b = O[:, :DETECTIONS_PER_IMG, 0:5]
    out_s = O[:, :DETECTIONS_PER_IMG, 5]
    out_l = jnp.round(O[:, :DETECTIONS_PER_IMG, 6]).astype(jnp.int32)
    return out_b, out_s, out_l


# speedup vs baseline: 1.0557x; 1.0557x over previous
import jax
import jax.numpy as jnp
from jax.experimental import pallas as pl
from jax.experimental.pallas import tpu as pltpu

SCORE_THRESH = 0.05
NMS_THRESH = 0.5
DETECTIONS_PER_IMG = 300
TOPK_CANDIDATES = 1000
NEG = -1e9

K = 1024
P = 384


def _corners(cx, cy, w, h, t):
    c = jnp.abs(jnp.cos(t))
    s = jnp.abs(jnp.sin(t))
    W = w * c + h * s
    H = w * s + h * c
    return cx - W * 0.5, cy - H * 0.5, cx + W * 0.5, cy + H * 0.5


def _nms_kernel(acol_ref, arow_ref, o_ref, iou_ref):
    Ac = acol_ref[0]
    Ar = arow_ref[0]

    x1c, y1c, x2c, y2c = _corners(
        Ac[:, 0:1], Ac[:, 1:2], Ac[:, 2:3], Ac[:, 3:4], Ac[:, 4:5])
    area_c = (x2c - x1c) * (y2c - y1c)
    x1r, y1r, x2r, y2r = _corners(
        Ar[0:1, :], Ar[1:2, :], Ar[2:3, :], Ar[3:4, :], Ar[4:5, :])
    area_r = (x2r - x1r) * (y2r - y1r)

    xx1 = jnp.maximum(x1c, x1r)
    yy1 = jnp.maximum(y1c, y1r)
    xx2 = jnp.minimum(x2c, x2r)
    yy2 = jnp.minimum(y2c, y2r)
    inter = jnp.maximum(xx2 - xx1, 0.0) * jnp.maximum(yy2 - yy1, 0.0)
    union = area_c + area_r - inter
    iou = inter / (union + 1e-7)
    same = Ac[:, 6:7] == Ar[6:7, :]
    iou_ref[:, :, :] = jnp.where(same, iou, 0.0).reshape(K, 8, 128)

    sub = jax.lax.broadcasted_iota(jnp.int32, (8, 128), 0)
    lane = jax.lax.broadcasted_iota(jnp.int32, (8, 128), 1)
    colid = sub * 128 + lane
    keep0 = jnp.ones((8, 128), jnp.float32)

    def body(i, keep):
        row = iou_ref[i]
        ki = jnp.sum(jnp.where(colid == i, keep, 0.0))
        sup = (row > NMS_THRESH) & (colid > i) & (ki > 0.5)
        return jnp.where(sup, 0.0, keep)

    keep = jax.lax.fori_loop(0, K, body, keep0)

    s_row = Ar[5:6, :]
    validf = keep.reshape(1, K) * (s_row > NEG / 2).astype(jnp.float32)

    ri = jax.lax.broadcasted_iota(jnp.int32, (K, K), 0)
    ci = jax.lax.broadcasted_iota(jnp.int32, (K, K), 1)
    tri = (ri <= ci).astype(jnp.float32)
    cum = jnp.dot(validf, tri, preferred_element_type=jnp.float32)
    pos = cum - 1.0

    prow = jax.lax.broadcasted_iota(jnp.int32, (P, 1), 0).astype(jnp.float32)
    onehot = ((prow == pos) & (validf > 0.5)).astype(jnp.float32)

    lab_p1 = Ac[:, 6:7] + 1.0
    vals = jnp.concatenate([Ac[:, 0:6], lab_p1, Ac[:, 7:8]], axis=1)
    out = jnp.dot(onehot, vals, preferred_element_type=jnp.float32)
    chan = jax.lax.broadcasted_iota(jnp.int32, (P, 8), 1)
    out = jnp.where(chan == 6, out - 1.0, out)
    o_ref[0] = out


def kernel(boxes, scores, labels):
    B, N = scores.shape
    scores_f = jnp.where(scores > SCORE_THRESH, scores, NEG)
    top_s, idx = jax.lax.top_k(scores_f, TOPK_CANDIDATES)
    top_b = jnp.take_along_axis(boxes, idx[..., None], axis=1)
    top_l = jnp.take_along_axis(labels, idx, axis=1)

    pad = K - TOPK_CANDIDATES
    top_s = jnp.pad(top_s, ((0, 0), (0, pad)), constant_values=NEG)
    top_b = jnp.pad(top_b, ((0, 0), (0, pad), (0, 0)))
    top_l = jnp.pad(top_l, ((0, 0), (0, pad)), constant_values=-1)

    A = jnp.concatenate(
        [top_b, top_s[..., None], top_l[..., None].astype(jnp.float32),
         jnp.zeros((B, K, 1), jnp.float32)], axis=-1)
    At = jnp.swapaxes(A, 1, 2)

    O = pl.pallas_call(
        _nms_kernel,
        grid=(B,),
        in_specs=[
            pl.BlockSpec((1, K, 8), lambda b: (b, 0, 0)),
            pl.BlockSpec((1, 8, K), lambda b: (b, 0, 0)),
        ],
        out_specs=pl.BlockSpec((1, P, 8), lambda b: (b, 0, 0)),
        out_shape=jax.ShapeDtypeStruct((B, P, 8), jnp.float32),
        scratch_shapes=[pltpu.VMEM((K, 8, 128), jnp.float32)],
        compiler_params=pltpu.CompilerParams(
            dimension_semantics=("parallel",)),
    )(A, At)

    out_b = O[:, :DETECTIONS_PER_IMG, 0:5]
    out_s = O[:, :DETECTIONS_PER_IMG, 5]
    out_l = jnp.round(O[:, :DETECTIONS_PER_IMG, 6]).astype(jnp.int32)
    return out_b, out_s, out_l


# speedup vs baseline: 1.0718x; 1.0153x over previous
import jax
import jax.numpy as jnp
from jax.experimental import pallas as pl
from jax.experimental.pallas import tpu as pltpu

SCORE_THRESH = 0.05
NMS_THRESH = 0.5
DETECTIONS_PER_IMG = 300
TOPK_CANDIDATES = 1000
NEG = -1e9

K = 1024
P = 384


def _corners(cx, cy, w, h, t):
    c = jnp.abs(jnp.cos(t))
    s = jnp.abs(jnp.sin(t))
    W = w * c + h * s
    H = w * s + h * c
    return cx - W * 0.5, cy - H * 0.5, cx + W * 0.5, cy + H * 0.5


def _nms_kernel(acol_ref, arow_ref, o_ref, iou_ref):
    Ac = acol_ref[0]
    Ar = arow_ref[0]

    x1c, y1c, x2c, y2c = _corners(
        Ac[:, 0:1], Ac[:, 1:2], Ac[:, 2:3], Ac[:, 3:4], Ac[:, 4:5])
    area_c = (x2c - x1c) * (y2c - y1c)
    x1r, y1r, x2r, y2r = _corners(
        Ar[0:1, :], Ar[1:2, :], Ar[2:3, :], Ar[3:4, :], Ar[4:5, :])
    area_r = (x2r - x1r) * (y2r - y1r)

    xx1 = jnp.maximum(x1c, x1r)
    yy1 = jnp.maximum(y1c, y1r)
    xx2 = jnp.minimum(x2c, x2r)
    yy2 = jnp.minimum(y2c, y2r)
    inter = jnp.maximum(xx2 - xx1, 0.0) * jnp.maximum(yy2 - yy1, 0.0)
    union = area_c + area_r - inter
    iou = inter / (union + 1e-7)
    same = Ac[:, 6:7] == Ar[6:7, :]
    iou_ref[:, :, :] = jnp.where(same, iou, 0.0).reshape(K, 8, 128)

    sub = jax.lax.broadcasted_iota(jnp.int32, (8, 128), 0)
    lane = jax.lax.broadcasted_iota(jnp.int32, (8, 128), 1)
    colid = sub * 128 + lane
    keep0 = jnp.ones((8, 128), jnp.float32)

    def body(i, keep):
        row = iou_ref[i]
        ki = jnp.sum(jnp.where(colid == i, keep, 0.0))
        sup = (row > NMS_THRESH) & (colid > i) & (ki > 0.5)
        return jnp.where(sup, 0.0, keep)

    keep = jax.lax.fori_loop(0, K, body, keep0, unroll=8)

    s_row = Ar[5:6, :]
    validf = keep.reshape(1, K) * (s_row > NEG / 2).astype(jnp.float32)

    ri = jax.lax.broadcasted_iota(jnp.int32, (K, K), 0)
    ci = jax.lax.broadcasted_iota(jnp.int32, (K, K), 1)
    tri = (ri <= ci).astype(jnp.float32)
    cum = jnp.dot(validf, tri, preferred_element_type=jnp.float32)
    pos = cum - 1.0

    prow = jax.lax.broadcasted_iota(jnp.int32, (P, 1), 0).astype(jnp.float32)
    onehot = ((prow == pos) & (validf > 0.5)).astype(jnp.float32)

    lab_p1 = Ac[:, 6:7] + 1.0
    vals = jnp.concatenate([Ac[:, 0:6], lab_p1, Ac[:, 7:8]], axis=1)
    out = jnp.dot(onehot, vals, preferred_element_type=jnp.float32)
    chan = jax.lax.broadcasted_iota(jnp.int32, (P, 8), 1)
    out = jnp.where(chan == 6, out - 1.0, out)
    o_ref[0] = out


def kernel(boxes, scores, labels):
    B, N = scores.shape
    scores_f = jnp.where(scores > SCORE_THRESH, scores, NEG)
    top_s, idx = jax.lax.top_k(scores_f, TOPK_CANDIDATES)
    top_b = jnp.take_along_axis(boxes, idx[..., None], axis=1)
    top_l = jnp.take_along_axis(labels, idx, axis=1)

    pad = K - TOPK_CANDIDATES
    top_s = jnp.pad(top_s, ((0, 0), (0, pad)), constant_values=NEG)
    top_b = jnp.pad(top_b, ((0, 0), (0, pad), (0, 0)))
    top_l = jnp.pad(top_l, ((0, 0), (0, pad)), constant_values=-1)

    A = jnp.concatenate(
        [top_b, top_s[..., None], top_l[..., None].astype(jnp.float32),
         jnp.zeros((B, K, 1), jnp.float32)], axis=-1)
    At = jnp.swapaxes(A, 1, 2)

    O = pl.pallas_call(
        _nms_kernel,
        grid=(B,),
        in_specs=[
            pl.BlockSpec((1, K, 8), lambda b: (b, 0, 0)),
            pl.BlockSpec((1, 8, K), lambda b: (b, 0, 0)),
        ],
        out_specs=pl.BlockSpec((1, P, 8), lambda b: (b, 0, 0)),
        out_shape=jax.ShapeDtypeStruct((B, P, 8), jnp.float32),
        scratch_shapes=[pltpu.VMEM((K, 8, 128), jnp.float32)],
        compiler_params=pltpu.CompilerParams(
            dimension_semantics=("parallel",)),
    )(A, At)

    out_b = O[:, :DETECTIONS_PER_IMG, 0:5]
    out_s = O[:, :DETECTIONS_PER_IMG, 5]
    out_l = jnp.round(O[:, :DETECTIONS_PER_IMG, 6]).astype(jnp.int32)
    return out_b, out_s, out_l
